# Initial kernel scaffold; baseline (speedup 1.0000x reference)
import jax
import jax.numpy as jnp
from jax.experimental import pallas as pl


def kernel(x, edge_index_near, edge_index_far, W0_near, b0_near, W0_far, b0_far, W1_near, b1_near, W1_far, b1_far, Wc1, bc1, Wc2, bc2):
    raise NotImplementedError("write your pallas kernel here")


# speedup vs baseline: 4.7564x; 4.7564x over previous
import jax
import jax.numpy as jnp
from jax import lax
from jax.experimental import pallas as pl
from jax.experimental.pallas import tpu as pltpu
from jax.experimental.pallas import tpu_sc as plsc

NC = 2
NS = 16
L = 16
C = 128
G = 40


def _mesh():
  return plsc.VectorSubcoreMesh(
      core_axis_name="c", subcore_axis_name="s", num_cores=NC,
      num_subcores=NS)


def _deg_kernel(dst_n, dst_f, N):
  E = dst_n.shape[0]
  n_chunks = E // C

  def body(dn_hbm, df_hbm, outn_hbm, outf_hbm, ones_v, idx_v, zero_v,
           acc_sh):
    c = lax.axis_index("c")
    s = lax.axis_index("s")

    for j in range(C // L):
      ones_v[pl.ds(j * L, L)] = jnp.ones((L,), jnp.float32)

    @pl.when(s == 0)
    def _():
      def zfill(i, carry):
        zero_v[pl.ds(i * L, L)] = jnp.zeros((L,), jnp.float32)
        return carry
      lax.fori_loop(0, N // L, zfill, 0)
      pltpu.sync_copy(zero_v, acc_sh)

    plsc.subcore_barrier()

    nch = n_chunks // NS + jnp.where(s < n_chunks % NS, 1, 0)

    def run(dst_ref):
      def chunk(t, carry):
        cid = s + t * NS
        pltpu.sync_copy(dst_ref.at[pl.ds(cid * C, C)], idx_v)
        pltpu.sync_copy(ones_v, acc_sh.at[idx_v], add=True)
        return carry
      lax.fori_loop(0, nch, chunk, 0)

    @pl.when(c == 0)
    def _():
      run(dn_hbm)

    @pl.when(c == 1)
    def _():
      run(df_hbm)

    plsc.subcore_barrier()

    @pl.when((s == 0) & (c == 0))
    def _():
      pltpu.sync_copy(acc_sh, outn_hbm)

    @pl.when((s == 0) & (c == 1))
    def _():
      pltpu.sync_copy(acc_sh, outf_hbm)

  f = pl.kernel(
      body,
      out_type=[
          jax.ShapeDtypeStruct((N,), jnp.float32),
          jax.ShapeDtypeStruct((N,), jnp.float32),
      ],
      mesh=_mesh(),
      scratch_types=[
          pltpu.VMEM((C,), jnp.float32),
          pltpu.VMEM((C,), jnp.int32),
          pltpu.VMEM((N,), jnp.float32),
          pltpu.VMEM_SHARED((N,), jnp.float32),
      ],
  )
  return f(dst_n, dst_f)


def _agg_kernel(h, src_n, dst_n, src_f, dst_f):
  N, D = h.shape
  E = src_n.shape[0]
  n_chunks = E // C
  n_groups = N // G

  def body(h_hbm, sn_hbm, dn_hbm, sf_hbm, df_hbm, out_hbm, idx_s, idx_d,
           rows_v, zero_v, sem, acc_sh):
    c = lax.axis_index("c")
    s = lax.axis_index("s")

    def zfill(i, carry):
      for j in range(D // L):
        zero_v[i, pl.ds(j * L, L)] = jnp.zeros((L,), jnp.float32)
      return carry
    lax.fori_loop(0, G, zfill, 0)

    ngr = n_groups // NS + jnp.where(s < n_groups % NS, 1, 0)

    def zgroup(t, carry):
      gid = s + t * NS
      pltpu.sync_copy(zero_v, acc_sh.at[pl.ds(gid * G, G), :])
      return carry
    lax.fori_loop(0, ngr, zgroup, 0)

    plsc.subcore_barrier()

    nch = n_chunks // NS + jnp.where(s < n_chunks % NS, 1, 0)

    def run(src_ref, dst_ref):
      def chunk(t, carry):
        cid = s + t * NS
        pltpu.sync_copy(src_ref.at[pl.ds(cid * C, C)], idx_s)
        pltpu.sync_copy(dst_ref.at[pl.ds(cid * C, C)], idx_d)
        pltpu.async_copy(h_hbm.at[idx_s], rows_v, sem).wait()
        pltpu.sync_copy(rows_v, acc_sh.at[idx_d], add=True)
        return carry
      lax.fori_loop(0, nch, chunk, 0)

    @pl.when(c == 0)
    def _():
      run(sn_hbm, dn_hbm)

    @pl.when(c == 1)
    def _():
      run(sf_hbm, df_hbm)

    plsc.subcore_barrier()

    def wgroup(t, carry):
      gid = s + t * NS
      pltpu.sync_copy(acc_sh.at[pl.ds(gid * G, G), :],
                      out_hbm.at[c, pl.ds(gid * G, G), :])
      return carry
    lax.fori_loop(0, ngr, wgroup, 0)

  f = pl.kernel(
      body,
      out_type=jax.ShapeDtypeStruct((NC, N, D), jnp.float32),
      mesh=_mesh(),
      scratch_types=[
          pltpu.VMEM((C,), jnp.int32),
          pltpu.VMEM((C,), jnp.int32),
          pltpu.VMEM((C, D), jnp.float32),
          pltpu.VMEM((G, D), jnp.float32),
          pltpu.SemaphoreType.DMA,
          pltpu.VMEM_SHARED((N, D), jnp.float32),
      ],
  )
  return f(h, src_n, dst_n, src_f, dst_f)


def _conv_tc(acc, deg_n, deg_f, Wn, bn, Wf, bf):
  _, N, D = acc.shape
  R = 1000
  grid = (N // R,)

  def body(acc_ref, dn_ref, df_ref, wn_ref, bn_ref, wf_ref, bf_ref, o_ref):
    an = acc_ref[0]
    af = acc_ref[1]
    dn = jnp.clip(dn_ref[...], 1.0, None)
    df = jnp.clip(df_ref[...], 1.0, None)
    hn = jnp.maximum(
        jnp.dot(an / dn, wn_ref[...], preferred_element_type=jnp.float32)
        + bn_ref[...], 0.0)
    hf = jnp.maximum(
        jnp.dot(af / df, wf_ref[...], preferred_element_type=jnp.float32)
        + bf_ref[...], 0.0)
    o_ref[...] = (hn + hf) * 0.5

  return pl.pallas_call(
      body,
      grid=grid,
      in_specs=[
          pl.BlockSpec((2, R, D), lambda i: (0, i, 0)),
          pl.BlockSpec((R, 1), lambda i: (i, 0)),
          pl.BlockSpec((R, 1), lambda i: (i, 0)),
          pl.BlockSpec((D, D), lambda i: (0, 0)),
          pl.BlockSpec((1, D), lambda i: (0, 0)),
          pl.BlockSpec((D, D), lambda i: (0, 0)),
          pl.BlockSpec((1, D), lambda i: (0, 0)),
      ],
      out_specs=pl.BlockSpec((R, D), lambda i: (i, 0)),
      out_shape=jax.ShapeDtypeStruct((N, D), jnp.float32),
  )(acc, deg_n, deg_f, Wn, bn, Wf, bf)


def _conv_cls_tc(acc, deg_n, deg_f, Wn, bn, Wf, bf, Wc1, bc1, Wc2, bc2):
  _, N, D = acc.shape
  H = Wc1.shape[1]
  O = Wc2.shape[1]
  R = 1000
  grid = (N // R,)

  def body(acc_ref, dn_ref, df_ref, wn_ref, bn_ref, wf_ref, bf_ref,
           wc1_ref, bc1_ref, wc2_ref, bc2_ref, o_ref):
    an = acc_ref[0]
    af = acc_ref[1]
    dn = jnp.clip(dn_ref[...], 1.0, None)
    df = jnp.clip(df_ref[...], 1.0, None)
    hn = jnp.maximum(
        jnp.dot(an / dn, wn_ref[...], preferred_element_type=jnp.float32)
        + bn_ref[...], 0.0)
    hf = jnp.maximum(
        jnp.dot(af / df, wf_ref[...], preferred_element_type=jnp.float32)
        + bf_ref[...], 0.0)
    h = (hn + hf) * 0.5
    t = jnp.dot(h, wc1_ref[...], preferred_element_type=jnp.float32) \
        + bc1_ref[...]
    o_ref[...] = jnp.dot(t, wc2_ref[...],
                         preferred_element_type=jnp.float32) + bc2_ref[...]

  return pl.pallas_call(
      body,
      grid=grid,
      in_specs=[
          pl.BlockSpec((2, R, D), lambda i: (0, i, 0)),
          pl.BlockSpec((R, 1), lambda i: (i, 0)),
          pl.BlockSpec((R, 1), lambda i: (i, 0)),
          pl.BlockSpec((D, D), lambda i: (0, 0)),
          pl.BlockSpec((1, D), lambda i: (0, 0)),
          pl.BlockSpec((D, D), lambda i: (0, 0)),
          pl.BlockSpec((1, D), lambda i: (0, 0)),
          pl.BlockSpec((D, H), lambda i: (0, 0)),
          pl.BlockSpec((1, H), lambda i: (0, 0)),
          pl.BlockSpec((H, O), lambda i: (0, 0)),
          pl.BlockSpec((1, O), lambda i: (0, 0)),
      ],
      out_specs=pl.BlockSpec((R, O), lambda i: (i, 0)),
      out_shape=jax.ShapeDtypeStruct((N, O), jnp.float32),
  )(acc, deg_n, deg_f, Wn, bn, Wf, bf, Wc1, bc1, Wc2, bc2)


def kernel(x, edge_index_near, edge_index_far,
           W0_near, b0_near, W0_far, b0_far,
           W1_near, b1_near, W1_far, b1_far,
           Wc1, bc1, Wc2, bc2):
  N = x.shape[0]
  sn, dn = edge_index_near[0], edge_index_near[1]
  sf, df = edge_index_far[0], edge_index_far[1]

  deg_n, deg_f = _deg_kernel(dn, df, N)
  deg_n = deg_n.reshape(N, 1)
  deg_f = deg_f.reshape(N, 1)

  b0n = b0_near.reshape(1, -1)
  b0f = b0_far.reshape(1, -1)
  b1n = b1_near.reshape(1, -1)
  b1f = b1_far.reshape(1, -1)
  bc1r = bc1.reshape(1, -1)
  bc2r = bc2.reshape(1, -1)

  acc0 = _agg_kernel(x, sn, dn, sf, df)
  h1 = _conv_tc(acc0, deg_n, deg_f, W0_near, b0n, W0_far, b0f)
  acc1 = _agg_kernel(h1, sn, dn, sf, df)
  out = _conv_cls_tc(acc1, deg_n, deg_f, W1_near, b1n, W1_far, b1f,
                     Wc1, bc1r, Wc2, bc2r)
  return out
